# baseline (speedup 1.0000x reference)
import functools

import jax
import jax.numpy as jnp
from jax.experimental import pallas as pl
from jax.experimental.pallas import tpu as pltpu


CODEBOOK_SIZE = 1024
CODE_DIM = 64
COMMITMENT_WEIGHT = 0.25


def _vq_kernel(x_ref, emb_ref, q_ref, idx_ref, loss_ref, esq_ref, row_ref):
    t_step = pl.program_id(1)
    xt = x_ref[0]
    emb = emb_ref[...]

    @pl.when(t_step == 0)
    def _per_batch_init():
        esq_ref[...] = jnp.sum(emb * emb, axis=1, keepdims=True)
        loss_ref[...] = jnp.zeros_like(loss_ref)
        row_ref[...] = jax.lax.broadcasted_iota(
            jnp.int32, row_ref.shape, 0)

    x16 = xt.astype(jnp.bfloat16)
    scores2 = jax.lax.dot_general(
        emb + emb, x16, (((1,), (0,)), ((), ())),
        preferred_element_type=jnp.float32)
    x_sq = jax.lax.dot_general(
        jnp.ones((8, CODE_DIM), jnp.float32), xt * xt,
        (((1,), (0,)), ((), ())),
        preferred_element_type=jnp.float32,
        precision=jax.lax.Precision.HIGHEST)[:1]
    dist = x_sq - scores2 + esq_ref[...]

    row = row_ref[...]
    m = jnp.min(dist, axis=0, keepdims=True)
    cand = jnp.where(dist == m, row, jnp.int32(CODEBOOK_SIZE))
    idx = jnp.min(cand, axis=0)
    idx_ref[0, 0] = idx

    onehot = (row == idx[None, :]).astype(jnp.bfloat16)
    q = jax.lax.dot_general(
        emb, onehot, (((0,), (0,)), ((), ())),
        preferred_element_type=jnp.float32)

    diff = q - xt
    q_ref[0] = xt + diff
    loss_ref[...] += jnp.sum(diff * diff, keepdims=True).reshape(1, 1, 1)


def kernel(x, embedding_weight):
    bsz, channels, steps = x.shape
    T = 1024
    grid = (bsz, steps // T)

    q, idx3, loss = pl.pallas_call(
        _vq_kernel,
        grid=grid,
        in_specs=[
            pl.BlockSpec((1, channels, T), lambda b, t: (b, 0, t)),
            pl.BlockSpec((CODEBOOK_SIZE, CODE_DIM), lambda b, t: (0, 0)),
        ],
        out_specs=[
            pl.BlockSpec((1, channels, T), lambda b, t: (b, 0, t)),
            pl.BlockSpec((1, 1, T), lambda b, t: (b, 0, t)),
            pl.BlockSpec((1, 1, 1), lambda b, t: (b, 0, 0)),
        ],
        out_shape=[
            jax.ShapeDtypeStruct((bsz, channels, steps), jnp.float32),
            jax.ShapeDtypeStruct((bsz, 1, steps), jnp.int32),
            jax.ShapeDtypeStruct((bsz, 1, 1), jnp.float32),
        ],
        scratch_shapes=[pltpu.VMEM((CODEBOOK_SIZE, 1), jnp.float32),
                        pltpu.VMEM((CODEBOOK_SIZE, T), jnp.int32)],
        compiler_params=pltpu.CompilerParams(
            dimension_semantics=("parallel", "arbitrary")),
    )(x, embedding_weight)

    scale = (1.0 + COMMITMENT_WEIGHT) / x.size
    return q, idx3.reshape(bsz, steps), (jnp.sum(loss) * scale).astype(jnp.float32)


# speedup vs baseline: 1.4598x; 1.4598x over previous
import functools

import jax
import jax.numpy as jnp
from jax.experimental import pallas as pl
from jax.experimental.pallas import tpu as pltpu


CODEBOOK_SIZE = 1024
CODE_DIM = 64
COMMITMENT_WEIGHT = 0.25


def _vq_kernel(x_ref, emb_ref, q_ref, idx_ref, loss_ref, esq_ref):
    t_step = pl.program_id(1)
    xt = x_ref[0]
    emb = emb_ref[...]

    @pl.when(t_step == 0)
    def _per_batch_init():
        esq_ref[...] = jnp.sum(emb * emb, axis=1, keepdims=True)
        loss_ref[...] = jnp.zeros_like(loss_ref)

    x16 = xt.astype(jnp.bfloat16)
    scores2 = jax.lax.dot_general(
        emb + emb, x16, (((1,), (0,)), ((), ())),
        preferred_element_type=jnp.float32)
    x_sq = jnp.sum(xt * xt, axis=0, keepdims=True)
    dist = x_sq - scores2 + esq_ref[...]

    row = jax.lax.broadcasted_iota(jnp.int32, dist.shape, 0)
    m = jnp.min(dist, axis=0, keepdims=True)
    cand = jnp.where(dist == m, row, jnp.int32(CODEBOOK_SIZE))
    idx = jnp.min(cand, axis=0)
    idx_ref[0, 0] = idx

    onehot = (row == idx[None, :]).astype(jnp.bfloat16)
    q = jax.lax.dot_general(
        emb, onehot, (((0,), (0,)), ((), ())),
        preferred_element_type=jnp.float32)

    diff = q - xt
    q_ref[0] = xt + diff
    loss_ref[...] += jnp.sum(diff * diff, keepdims=True).reshape(1, 1, 1)


def kernel(x, embedding_weight):
    bsz, channels, steps = x.shape
    T = 2048
    grid = (bsz, steps // T)

    q, idx3, loss = pl.pallas_call(
        _vq_kernel,
        grid=grid,
        in_specs=[
            pl.BlockSpec((1, channels, T), lambda b, t: (b, 0, t)),
            pl.BlockSpec((CODEBOOK_SIZE, CODE_DIM), lambda b, t: (0, 0)),
        ],
        out_specs=[
            pl.BlockSpec((1, channels, T), lambda b, t: (b, 0, t)),
            pl.BlockSpec((1, 1, T), lambda b, t: (b, 0, t)),
            pl.BlockSpec((1, 1, 1), lambda b, t: (b, 0, 0)),
        ],
        out_shape=[
            jax.ShapeDtypeStruct((bsz, channels, steps), jnp.float32),
            jax.ShapeDtypeStruct((bsz, 1, steps), jnp.int32),
            jax.ShapeDtypeStruct((bsz, 1, 1), jnp.float32),
        ],
        scratch_shapes=[pltpu.VMEM((CODEBOOK_SIZE, 1), jnp.float32)],
        compiler_params=pltpu.CompilerParams(
            dimension_semantics=("parallel", "arbitrary")),
    )(x, embedding_weight)

    scale = (1.0 + COMMITMENT_WEIGHT) / x.size
    return q, idx3.reshape(bsz, steps), (jnp.sum(loss) * scale).astype(jnp.float32)


# speedup vs baseline: 1.5285x; 1.0471x over previous
import functools

import jax
import jax.numpy as jnp
from jax.experimental import pallas as pl
from jax.experimental.pallas import tpu as pltpu


CODEBOOK_SIZE = 1024
CODE_DIM = 64
COMMITMENT_WEIGHT = 0.25


def _vq_kernel(x_ref, emb_ref, q_ref, idx_ref, loss_ref, esq_ref):
    t_step = pl.program_id(1)
    xt = x_ref[0]
    emb = emb_ref[...]

    @pl.when(t_step == 0)
    def _per_batch_init():
        esq_ref[...] = jnp.sum(emb * emb, axis=1, keepdims=True)
        loss_ref[...] = jnp.zeros_like(loss_ref)

    x16 = xt.astype(jnp.bfloat16)
    scores2 = jax.lax.dot_general(
        emb + emb, x16, (((1,), (0,)), ((), ())),
        preferred_element_type=jnp.float32)
    x_sq = jnp.sum(xt * xt, axis=0, keepdims=True)
    dist = x_sq - scores2 + esq_ref[...]

    row = jax.lax.broadcasted_iota(jnp.int32, dist.shape, 0)
    m = jnp.min(dist, axis=0, keepdims=True)
    cand = jnp.where(dist == m, row, jnp.int32(CODEBOOK_SIZE))
    idx = jnp.min(cand, axis=0)
    idx_ref[0, 0] = idx

    onehot = (row == idx[None, :]).astype(jnp.bfloat16)
    q = jax.lax.dot_general(
        emb, onehot, (((0,), (0,)), ((), ())),
        preferred_element_type=jnp.float32)

    diff = q - xt
    q_ref[0] = xt + diff
    loss_ref[...] += jnp.sum(diff * diff, keepdims=True).reshape(1, 1, 1)


def kernel(x, embedding_weight):
    bsz, channels, steps = x.shape
    T = 4096
    grid = (bsz, steps // T)

    q, idx3, loss = pl.pallas_call(
        _vq_kernel,
        grid=grid,
        in_specs=[
            pl.BlockSpec((1, channels, T), lambda b, t: (b, 0, t)),
            pl.BlockSpec((CODEBOOK_SIZE, CODE_DIM), lambda b, t: (0, 0)),
        ],
        out_specs=[
            pl.BlockSpec((1, channels, T), lambda b, t: (b, 0, t)),
            pl.BlockSpec((1, 1, T), lambda b, t: (b, 0, t)),
            pl.BlockSpec((1, 1, 1), lambda b, t: (b, 0, 0)),
        ],
        out_shape=[
            jax.ShapeDtypeStruct((bsz, channels, steps), jnp.float32),
            jax.ShapeDtypeStruct((bsz, 1, steps), jnp.int32),
            jax.ShapeDtypeStruct((bsz, 1, 1), jnp.float32),
        ],
        scratch_shapes=[pltpu.VMEM((CODEBOOK_SIZE, 1), jnp.float32)],
        compiler_params=pltpu.CompilerParams(
            dimension_semantics=("parallel", "arbitrary")),
    )(x, embedding_weight)

    scale = (1.0 + COMMITMENT_WEIGHT) / x.size
    return q, idx3.reshape(bsz, steps), (jnp.sum(loss) * scale).astype(jnp.float32)


# speedup vs baseline: 1.6268x; 1.0643x over previous
import functools

import jax
import jax.numpy as jnp
from jax.experimental import pallas as pl
from jax.experimental.pallas import tpu as pltpu


CODEBOOK_SIZE = 1024
CODE_DIM = 64
COMMITMENT_WEIGHT = 0.25


def _vq_kernel(x_ref, emb_ref, q_ref, idx_ref, loss_ref, esq_ref):
    t_step = pl.program_id(1)
    xt = x_ref[0]
    emb = emb_ref[...]

    @pl.when(t_step == 0)
    def _per_batch_init():
        esq_ref[...] = jnp.sum(emb * emb, axis=1, keepdims=True)
        loss_ref[...] = jnp.zeros_like(loss_ref)

    x16 = xt.astype(jnp.bfloat16)
    scores2 = jax.lax.dot_general(
        emb + emb, x16, (((1,), (0,)), ((), ())),
        preferred_element_type=jnp.float32)
    x_sq = jnp.sum(xt * xt, axis=0, keepdims=True)
    dist = x_sq - scores2 + esq_ref[...]

    row = jax.lax.broadcasted_iota(
        jnp.int32, dist.shape, 0).astype(jnp.float32)
    m = jnp.min(dist, axis=0, keepdims=True)
    cand = jnp.where(dist == m, row, jnp.float32(CODEBOOK_SIZE))
    idx = jnp.min(cand, axis=0, keepdims=True)
    idx_ref[0] = idx.astype(jnp.int32)

    onehot = (row == idx).astype(jnp.bfloat16)
    q = jax.lax.dot_general(
        emb, onehot, (((0,), (0,)), ((), ())),
        preferred_element_type=jnp.float32)

    diff = q - xt
    q_ref[0] = xt + diff
    loss_ref[...] += jnp.sum(diff * diff, keepdims=True).reshape(1, 1, 1)


def kernel(x, embedding_weight):
    bsz, channels, steps = x.shape
    T = 4096
    grid = (bsz, steps // T)

    q, idx3, loss = pl.pallas_call(
        _vq_kernel,
        grid=grid,
        in_specs=[
            pl.BlockSpec((1, channels, T), lambda b, t: (b, 0, t)),
            pl.BlockSpec((CODEBOOK_SIZE, CODE_DIM), lambda b, t: (0, 0)),
        ],
        out_specs=[
            pl.BlockSpec((1, channels, T), lambda b, t: (b, 0, t)),
            pl.BlockSpec((1, 1, T), lambda b, t: (b, 0, t)),
            pl.BlockSpec((1, 1, 1), lambda b, t: (b, 0, 0)),
        ],
        out_shape=[
            jax.ShapeDtypeStruct((bsz, channels, steps), jnp.float32),
            jax.ShapeDtypeStruct((bsz, 1, steps), jnp.int32),
            jax.ShapeDtypeStruct((bsz, 1, 1), jnp.float32),
        ],
        scratch_shapes=[pltpu.VMEM((CODEBOOK_SIZE, 1), jnp.float32)],
        compiler_params=pltpu.CompilerParams(
            dimension_semantics=("parallel", "arbitrary")),
    )(x, embedding_weight)

    scale = (1.0 + COMMITMENT_WEIGHT) / x.size
    return q, idx3.reshape(bsz, steps), (jnp.sum(loss) * scale).astype(jnp.float32)


# speedup vs baseline: 1.6753x; 1.0298x over previous
import functools

import jax
import jax.numpy as jnp
from jax.experimental import pallas as pl
from jax.experimental.pallas import tpu as pltpu


CODEBOOK_SIZE = 1024
CODE_DIM = 64
COMMITMENT_WEIGHT = 0.25


def _vq_kernel(x_ref, emb_ref, q_ref, idx_ref, loss_ref, esq_ref):
    t_step = pl.program_id(1)
    xt = x_ref[0]
    emb = emb_ref[...]

    @pl.when(t_step == 0)
    def _per_batch_init():
        esq_ref[...] = jnp.sum(emb * emb, axis=1, keepdims=True)
        loss_ref[...] = jnp.zeros_like(loss_ref)

    row = jax.lax.broadcasted_iota(
        jnp.int32, (CODEBOOK_SIZE, 1), 0).astype(jnp.float32)
    emb2 = emb + emb
    esq = esq_ref[...]

    T = xt.shape[1]
    H = T // 2
    loss_acc = jnp.zeros((1, 1), jnp.float32)
    for s in (0, 1):
        sl = slice(s * H, (s + 1) * H)
        xh = xt[:, sl]
        x16 = xh.astype(jnp.bfloat16)
        scores2 = jax.lax.dot_general(
            emb2, x16, (((1,), (0,)), ((), ())),
            preferred_element_type=jnp.float32)
        x_sq = jnp.sum(xh * xh, axis=0, keepdims=True)
        dist = x_sq - scores2 + esq

        m = jnp.min(dist, axis=0, keepdims=True)
        cand = jnp.where(dist == m, row, jnp.float32(CODEBOOK_SIZE))
        idx = jnp.min(cand, axis=0, keepdims=True)
        idx_ref[0, :, sl] = idx.astype(jnp.int32)

        onehot = (row == idx).astype(jnp.bfloat16)
        q = jax.lax.dot_general(
            emb, onehot, (((0,), (0,)), ((), ())),
            preferred_element_type=jnp.float32)

        diff = q - xh
        q_ref[0, :, sl] = xh + diff
        loss_acc = loss_acc + jnp.sum(diff * diff, keepdims=True)

    loss_ref[...] += loss_acc.reshape(1, 1, 1)


def kernel(x, embedding_weight):
    bsz, channels, steps = x.shape
    T = 4096
    grid = (bsz, steps // T)

    q, idx3, loss = pl.pallas_call(
        _vq_kernel,
        grid=grid,
        in_specs=[
            pl.BlockSpec((1, channels, T), lambda b, t: (b, 0, t)),
            pl.BlockSpec((CODEBOOK_SIZE, CODE_DIM), lambda b, t: (0, 0)),
        ],
        out_specs=[
            pl.BlockSpec((1, channels, T), lambda b, t: (b, 0, t)),
            pl.BlockSpec((1, 1, T), lambda b, t: (b, 0, t)),
            pl.BlockSpec((1, 1, 1), lambda b, t: (b, 0, 0)),
        ],
        out_shape=[
            jax.ShapeDtypeStruct((bsz, channels, steps), jnp.float32),
            jax.ShapeDtypeStruct((bsz, 1, steps), jnp.int32),
            jax.ShapeDtypeStruct((bsz, 1, 1), jnp.float32),
        ],
        scratch_shapes=[pltpu.VMEM((CODEBOOK_SIZE, 1), jnp.float32)],
        compiler_params=pltpu.CompilerParams(
            dimension_semantics=("parallel", "arbitrary")),
    )(x, embedding_weight)

    scale = (1.0 + COMMITMENT_WEIGHT) / x.size
    return q, idx3.reshape(bsz, steps), (jnp.sum(loss) * scale).astype(jnp.float32)
